# Initial kernel scaffold; baseline (speedup 1.0000x reference)
import jax
import jax.numpy as jnp
from jax.experimental import pallas as pl


def kernel(tokens, W_pos):
    raise NotImplementedError("write your pallas kernel here")


# speedup vs baseline: 1.2004x; 1.2004x over previous
import jax
import jax.numpy as jnp
from jax.experimental import pallas as pl


def _copy_body(w_ref, o_ref):
    o_ref[0] = w_ref[...]


def kernel(tokens, W_pos):
    B, S = tokens.shape
    D = W_pos.shape[1]
    BS = 512

    grid = (S // BS, B)
    out = pl.pallas_call(
        _copy_body,
        grid=grid,
        in_specs=[pl.BlockSpec((BS, D), lambda i, b: (i, 0))],
        out_specs=pl.BlockSpec((1, BS, D), lambda i, b: (b, i, 0)),
        out_shape=jax.ShapeDtypeStruct((B, S, D), jnp.float32),
    )(W_pos)
    return out
